# baseline (speedup 1.0000x reference)
import functools

import jax
import jax.numpy as jnp
from jax import lax
from jax.experimental import pallas as pl
from jax.experimental.pallas import tpu as pltpu
from jax.experimental.pallas import tpu_sc as plsc

N_BINS = 128
L = 16
N_CHUNKS = N_BINS // L


def _loss_cdf_body(t_hbm, lt_hbm, lu_hbm, out_hbm,
                   lt_v, lu_v, et_v, eu_v, b_v, t_v, u_v, sem):
    nc = 2
    wid = lax.axis_index("s") * nc + lax.axis_index("c")
    n_tok = t_hbm.shape[0]
    slab = n_tok // 32
    base = wid * slab

    t_cp = pltpu.async_copy(t_hbm.at[pl.ds(base, slab)], t_v, sem)
    pltpu.sync_copy(lt_hbm, lt_v)
    pltpu.sync_copy(lu_hbm, lu_v)

    ex_t = [jnp.exp(lt_v[pl.ds(c * L, L)]) for c in range(N_CHUNKS)]
    ex_u = [jnp.exp(lu_v[pl.ds(c * L, L)]) for c in range(N_CHUNKS)]
    s_t = functools.reduce(lax.add, [jnp.sum(e) for e in ex_t])
    q_t = [e / s_t + 0.001 for e in ex_t]
    z_t = functools.reduce(lax.add, [jnp.sum(q) for q in q_t])
    q_u = [e + 0.001 for e in ex_u]
    z_u = functools.reduce(lax.add, [jnp.sum(q) for q in q_u])

    carry_t = jnp.float32(0.0)
    carry_u = jnp.float32(0.0)
    for c in range(N_CHUNKS):
        w_t = q_t[c] / z_t
        w_u = q_u[c] / z_u
        b_v[pl.ds(c * L, L)] = w_u / w_t
        incl_t = plsc.cumsum(w_t)
        incl_u = plsc.cumsum(w_u)
        et_v[pl.ds(c * L, L)] = incl_t - w_t + carry_t
        eu_v[pl.ds(c * L, L)] = incl_u - w_u + carry_u
        carry_t = carry_t + jnp.sum(w_t)
        carry_u = carry_u + jnp.sum(w_u)

    t_cp.wait()

    @plsc.parallel_loop(0, slab, L, unroll=8)
    def _token_loop(off):
        tv = t_v[pl.ds(off, L)]
        idx = jnp.zeros((L,), jnp.int32)
        e_t0 = jnp.zeros((L,), jnp.float32)
        for s in (64, 32, 16, 8, 4, 2, 1):
            cand = idx + s
            ev = plsc.load_gather(et_v, [cand])
            take = ev <= tv
            idx = jnp.where(take, cand, idx)
            e_t0 = jnp.where(take, ev, e_t0)
        e_u0 = plsc.load_gather(eu_v, [idx])
        bv = plsc.load_gather(b_v, [idx])
        u_v[pl.ds(off, L)] = e_u0 + bv * (tv - e_t0)

    pltpu.sync_copy(u_v, out_hbm.at[pl.ds(base, slab)])


def kernel(t, l_t, l_u):
    b, s = t.shape
    n_tok = b * s
    slab = n_tok // 32
    t_flat = t.reshape(n_tok)
    mesh = plsc.VectorSubcoreMesh(core_axis_name="c", subcore_axis_name="s")
    run = pl.kernel(
        _loss_cdf_body,
        mesh=mesh,
        compiler_params=pltpu.CompilerParams(needs_layout_passes=False),
        out_type=jax.ShapeDtypeStruct((n_tok,), jnp.float32),
        scratch_types=[
            pltpu.VMEM((N_BINS,), jnp.float32),
            pltpu.VMEM((N_BINS,), jnp.float32),
            pltpu.VMEM((N_BINS,), jnp.float32),
            pltpu.VMEM((N_BINS,), jnp.float32),
            pltpu.VMEM((N_BINS,), jnp.float32),
            pltpu.VMEM((slab,), jnp.float32),
            pltpu.VMEM((slab,), jnp.float32),
            pltpu.SemaphoreType.DMA,
        ],
    )
    u_flat = run(t_flat, l_t, l_u)
    return u_flat.reshape(b, s)


# speedup vs baseline: 1.0344x; 1.0344x over previous
import functools

import jax
import jax.numpy as jnp
from jax import lax
from jax.experimental import pallas as pl
from jax.experimental.pallas import tpu as pltpu
from jax.experimental.pallas import tpu_sc as plsc

N_BINS = 128
L = 16
N_CHUNKS = N_BINS // L


def _loss_cdf_body(t_hbm, lt_hbm, lu_hbm, out_hbm,
                   lt_v, lu_v, et_v, a_v, b_v, t_v, u_v, sem):
    nc = 2
    wid = lax.axis_index("s") * nc + lax.axis_index("c")
    n_tok = t_hbm.shape[0]
    slab = n_tok // 32
    base = wid * slab

    t_cp = pltpu.async_copy(t_hbm.at[pl.ds(base, slab)], t_v, sem)
    pltpu.sync_copy(lt_hbm, lt_v)
    pltpu.sync_copy(lu_hbm, lu_v)

    ex_t = [jnp.exp(lt_v[pl.ds(c * L, L)]) for c in range(N_CHUNKS)]
    ex_u = [jnp.exp(lu_v[pl.ds(c * L, L)]) for c in range(N_CHUNKS)]
    s_t = functools.reduce(lax.add, [jnp.sum(e) for e in ex_t])
    q_t = [e / s_t + 0.001 for e in ex_t]
    z_t = functools.reduce(lax.add, [jnp.sum(q) for q in q_t])
    q_u = [e + 0.001 for e in ex_u]
    z_u = functools.reduce(lax.add, [jnp.sum(q) for q in q_u])

    carry_t = jnp.float32(0.0)
    carry_u = jnp.float32(0.0)
    for c in range(N_CHUNKS):
        w_t = q_t[c] / z_t
        w_u = q_u[c] / z_u
        b = w_u / w_t
        incl_t = plsc.cumsum(w_t)
        incl_u = plsc.cumsum(w_u)
        e_t = incl_t - w_t + carry_t
        e_u = incl_u - w_u + carry_u
        b_v[pl.ds(c * L, L)] = b
        et_v[pl.ds(c * L, L)] = e_t
        a_v[pl.ds(c * L, L)] = e_u - b * e_t
        carry_t = carry_t + jnp.sum(w_t)
        carry_u = carry_u + jnp.sum(w_u)

    t_cp.wait()

    @plsc.parallel_loop(0, slab, L, unroll=4)
    def _token_loop(off):
        tv = t_v[pl.ds(off, L)]
        idx = jnp.zeros((L,), jnp.int32)
        for s in (64, 32, 16, 8, 4, 2, 1):
            cand = idx + s
            ev = plsc.load_gather(et_v, [cand])
            idx = jnp.where(ev <= tv, cand, idx)
        av = plsc.load_gather(a_v, [idx])
        bv = plsc.load_gather(b_v, [idx])
        u_v[pl.ds(off, L)] = av + bv * tv

    pltpu.sync_copy(u_v, out_hbm.at[pl.ds(base, slab)])


def kernel(t, l_t, l_u):
    b, s = t.shape
    n_tok = b * s
    slab = n_tok // 32
    t_flat = t.reshape(n_tok)
    mesh = plsc.VectorSubcoreMesh(core_axis_name="c", subcore_axis_name="s")
    run = pl.kernel(
        _loss_cdf_body,
        mesh=mesh,
        compiler_params=pltpu.CompilerParams(needs_layout_passes=False),
        out_type=jax.ShapeDtypeStruct((n_tok,), jnp.float32),
        scratch_types=[
            pltpu.VMEM((N_BINS,), jnp.float32),
            pltpu.VMEM((N_BINS,), jnp.float32),
            pltpu.VMEM((N_BINS,), jnp.float32),
            pltpu.VMEM((N_BINS,), jnp.float32),
            pltpu.VMEM((N_BINS,), jnp.float32),
            pltpu.VMEM((slab,), jnp.float32),
            pltpu.VMEM((slab,), jnp.float32),
            pltpu.SemaphoreType.DMA,
        ],
    )
    u_flat = run(t_flat, l_t, l_u)
    return u_flat.reshape(b, s)


# speedup vs baseline: 1.1155x; 1.0783x over previous
import functools

import jax
import jax.numpy as jnp
from jax import lax
from jax.experimental import pallas as pl
from jax.experimental.pallas import tpu as pltpu
from jax.experimental.pallas import tpu_sc as plsc

N_BINS = 128
L = 16
N_CHUNKS = N_BINS // L


def _loss_cdf_body(t_hbm, lt_hbm, lu_hbm, out_hbm,
                   lt_v, lu_v, et_v, a_v, b_v, t_v, u_v, sem):
    nc = 2
    wid = lax.axis_index("s") * nc + lax.axis_index("c")
    r0 = 8 * (wid // 16)
    c0 = 256 * (wid % 16)

    t_cp = pltpu.async_copy(
        t_hbm.at[pl.ds(r0, 8), pl.ds(c0, 256)], t_v, sem)
    pltpu.sync_copy(lt_hbm, lt_v)
    pltpu.sync_copy(lu_hbm, lu_v)

    ex_t = [jnp.exp(lt_v[pl.ds(c * L, L)]) for c in range(N_CHUNKS)]
    ex_u = [jnp.exp(lu_v[pl.ds(c * L, L)]) for c in range(N_CHUNKS)]
    s_t = functools.reduce(lax.add, [jnp.sum(e) for e in ex_t])
    q_t = [e / s_t + 0.001 for e in ex_t]
    z_t = functools.reduce(lax.add, [jnp.sum(q) for q in q_t])
    q_u = [e + 0.001 for e in ex_u]
    z_u = functools.reduce(lax.add, [jnp.sum(q) for q in q_u])

    carry_t = jnp.float32(0.0)
    carry_u = jnp.float32(0.0)
    for c in range(N_CHUNKS):
        w_t = q_t[c] / z_t
        w_u = q_u[c] / z_u
        b = w_u / w_t
        incl_t = plsc.cumsum(w_t)
        incl_u = plsc.cumsum(w_u)
        e_t = incl_t - w_t + carry_t
        e_u = incl_u - w_u + carry_u
        b_v[pl.ds(c * L, L)] = b
        et_v[pl.ds(c * L, L)] = e_t
        a_v[pl.ds(c * L, L)] = e_u - b * e_t
        carry_t = carry_t + jnp.sum(w_t)
        carry_u = carry_u + jnp.sum(w_u)

    t_cp.wait()

    @plsc.parallel_loop(0, 2048, L, unroll=4)
    def _token_loop(off):
        r = jax.lax.shift_right_logical(off, 8)
        col = jax.lax.bitwise_and(off, 255)
        tv = t_v[r, pl.ds(col, L)]
        idx = jnp.zeros((L,), jnp.int32)
        for s in (64, 32, 16, 8, 4, 2, 1):
            cand = idx + s
            ev = plsc.load_gather(et_v, [cand])
            idx = jnp.where(ev <= tv, cand, idx)
        av = plsc.load_gather(a_v, [idx])
        bv = plsc.load_gather(b_v, [idx])
        u_v[r, pl.ds(col, L)] = av + bv * tv

    pltpu.sync_copy(u_v, out_hbm.at[pl.ds(r0, 8), pl.ds(c0, 256)])


def kernel(t, l_t, l_u):
    b, s = t.shape
    mesh = plsc.VectorSubcoreMesh(core_axis_name="c", subcore_axis_name="s")
    run = pl.kernel(
        _loss_cdf_body,
        mesh=mesh,
        compiler_params=pltpu.CompilerParams(
            needs_layout_passes=False, use_tc_tiling_on_sc=True),
        out_type=jax.ShapeDtypeStruct((b, s), jnp.float32),
        scratch_types=[
            pltpu.VMEM((N_BINS,), jnp.float32),
            pltpu.VMEM((N_BINS,), jnp.float32),
            pltpu.VMEM((N_BINS,), jnp.float32),
            pltpu.VMEM((N_BINS,), jnp.float32),
            pltpu.VMEM((N_BINS,), jnp.float32),
            pltpu.VMEM((8, 256), jnp.float32),
            pltpu.VMEM((8, 256), jnp.float32),
            pltpu.SemaphoreType.DMA,
        ],
    )
    return run(t, l_t, l_u)
